# Initial kernel scaffold; baseline (speedup 1.0000x reference)
import functools

import jax
import jax.numpy as jnp
from jax import lax
from jax.experimental import pallas as pl
from jax.experimental.pallas import tpu as pltpu
from jax.experimental.pallas import tpu_sc as plsc

N = 10000
E = 320000
R = 8
F = 128
L = 16
NSLICE = F // L
NSEG = N * R
NC = 2
NS = 16
ROWLEN = 128
KROWS = -(-E // (NS * ROWLEN))
E_PAD = NS * ROWLEN * KROWS
ACC_ROWS = NSEG + L
ZROWS_TILE = ACC_ROWS // NS
ZCH = ZROWS_TILE // 3
SEG_TILE = NSEG // NS


def _make_sc_agg(with_cnt, interpret=False):
    outs = [jax.ShapeDtypeStruct((NSLICE, NSEG, L), jnp.float32)]
    if with_cnt:
        outs.append(jax.ShapeDtypeStruct((NSEG, L), jnp.float32))
    mesh = plsc.VectorSubcoreMesh(core_axis_name="c", subcore_axis_name="s")

    @functools.partial(
        pl.kernel,
        out_type=tuple(outs) if with_cnt else outs[0],
        mesh=mesh,
        interpret=interpret,
        scratch_types=[
            pltpu.VMEM_SHARED((ACC_ROWS, L), jnp.float32),
            pltpu.VMEM((KROWS, ROWLEN), jnp.int32),
            pltpu.VMEM((KROWS, ROWLEN), jnp.int32),
            pltpu.VMEM((ROWLEN, L), jnp.float32),
            pltpu.VMEM((ROWLEN, L), jnp.float32),
            pltpu.SemaphoreType.DMA,
        ],
    )
    def sc_kernel(xv_hbm, gidx_hbm, seg_hbm, ones_hbm, zeros_hbm, *rest):
        if with_cnt:
            a_out, cnt_out, acc_sh, seg_v, gi_v, row_v, ones_v, gsem = rest
        else:
            a_out, acc_sh, seg_v, gi_v, row_v, ones_v, gsem = rest
        c = lax.axis_index("c")
        s = lax.axis_index("s")

        pltpu.sync_copy(seg_hbm.at[s], seg_v)
        pltpu.sync_copy(ones_hbm, ones_v)

        def zero_my_stripe():
            for kz in range(3):
                pltpu.sync_copy(
                    zeros_hbm, acc_sh.at[pl.ds(s * ZROWS_TILE + kz * ZCH, ZCH)])

        for i in range(NSLICE // NC):
            pglob = c * (NSLICE // NC) + i
            zero_my_stripe()
            plsc.subcore_barrier()
            pltpu.sync_copy(gidx_hbm.at[pglob, s], gi_v)

            def ebody(j, carry):
                pltpu.async_copy(xv_hbm.at[gi_v.at[j]], row_v, gsem).wait()
                pltpu.sync_copy(row_v, acc_sh.at[seg_v.at[j]], add=True)
                return carry

            lax.fori_loop(0, KROWS, ebody, 0)
            plsc.subcore_barrier()
            pltpu.sync_copy(acc_sh.at[pl.ds(s * SEG_TILE, SEG_TILE)],
                            a_out.at[pglob, pl.ds(s * SEG_TILE, SEG_TILE)])
            plsc.subcore_barrier()

        if with_cnt:
            @pl.when(c == 1)
            def _():
                zero_my_stripe()
                plsc.subcore_barrier()

                def cbody(j, carry):
                    pltpu.sync_copy(ones_v, acc_sh.at[seg_v.at[j]], add=True)
                    return carry

                lax.fori_loop(0, KROWS, cbody, 0)
                plsc.subcore_barrier()
                pltpu.sync_copy(acc_sh.at[pl.ds(s * SEG_TILE, SEG_TILE)],
                                cnt_out.at[pl.ds(s * SEG_TILE, SEG_TILE)])

    return sc_kernel


def _tc_layer(a, cnt, xin, basis, comp, root, bias, act, interpret=False):
    out_dim = root.shape[1]
    w = jnp.einsum('rb,bio->rio', comp, basis)
    w2 = w.reshape(R, NSLICE, L, out_dim).transpose(1, 0, 2, 3).reshape(
        NSLICE, F, out_dim)
    a3 = a.reshape(NSLICE, N, F)
    cnt2 = cnt.reshape(N, F)
    bias2 = bias.reshape(1, out_dim)
    nb = 1250
    grid = (N // nb,)

    def body(a_ref, c_ref, x_ref, w2_ref, root_ref, b_ref, o_ref):
        recip = 1.0 / jnp.maximum(c_ref[...], 1.0)
        acc = jnp.dot(x_ref[...], root_ref[...],
                      preferred_element_type=jnp.float32)
        for p in range(NSLICE):
            acc += jnp.dot(a_ref[p] * recip, w2_ref[p],
                           preferred_element_type=jnp.float32)
        acc += b_ref[...]
        if act == 'relu':
            acc = jnp.maximum(acc, 0.0)
        else:
            acc = jnp.tanh(acc)
        o_ref[...] = acc

    return pl.pallas_call(
        body,
        grid=grid,
        in_specs=[
            pl.BlockSpec((NSLICE, nb, F), lambda i: (0, i, 0)),
            pl.BlockSpec((nb, F), lambda i: (i, 0)),
            pl.BlockSpec((nb, F), lambda i: (i, 0)),
            pl.BlockSpec((NSLICE, F, out_dim), lambda i: (0, 0, 0)),
            pl.BlockSpec((F, out_dim), lambda i: (0, 0)),
            pl.BlockSpec((1, out_dim), lambda i: (0, 0)),
        ],
        out_specs=pl.BlockSpec((nb, out_dim), lambda i: (i, 0)),
        out_shape=jax.ShapeDtypeStruct((N, out_dim), jnp.float32),
        interpret=interpret,
    )(a3, cnt2, xin, w2, root, bias2)


def kernel(x, edge_index, edge_type,
           basis0, comp0, root0, bias0,
           basis1, comp1, root1, bias1,
           basis2, comp2, root2, bias2):
    src = edge_index[0].astype(jnp.int32)
    dst = edge_index[1].astype(jnp.int32)
    seg = dst * R + edge_type.astype(jnp.int32)

    pad = E_PAD - E
    seg_p = jnp.concatenate(
        [seg, jnp.full((pad,), NSEG, jnp.int32)]).reshape(NS, KROWS, ROWLEN)
    src_p = jnp.concatenate([src, jnp.zeros((pad,), jnp.int32)])
    src_rs = src_p.reshape(NS, KROWS, ROWLEN)
    gidx = (src_rs[None] * NSLICE
            + jnp.arange(NSLICE, dtype=jnp.int32)[:, None, None, None])
    ones_rows = jnp.ones((ROWLEN, L), jnp.float32)
    zeros_buf = jnp.zeros((ZCH, L), jnp.float32)

    sc_first = _make_sc_agg(True)
    sc_rest = _make_sc_agg(False)

    a0, cnt = sc_first(x.reshape(N * NSLICE, L), gidx, seg_p,
                       ones_rows, zeros_buf)
    h0 = _tc_layer(a0, cnt, x, basis0, comp0, root0, bias0, 'relu')
    a1 = sc_rest(h0.reshape(N * NSLICE, L), gidx, seg_p, ones_rows, zeros_buf)
    h1 = _tc_layer(a1, cnt, h0, basis1, comp1, root1, bias1, 'relu')
    a2 = sc_rest(h1.reshape(N * NSLICE, L), gidx, seg_p, ones_rows, zeros_buf)
    return _tc_layer(a2, cnt, h1, basis2, comp2, root2, bias2, 'tanh')


# speedup vs baseline: 2.8700x; 2.8700x over previous
import functools

import jax
import jax.numpy as jnp
from jax import lax
from jax.experimental import pallas as pl
from jax.experimental.pallas import tpu as pltpu
from jax.experimental.pallas import tpu_sc as plsc

N = 10000
E = 320000
R = 8
F = 128
L = 16
NSLICE = F // L
NSEG = N * R
NC = 2
NS = 16
ROWLEN = 128
KROWS = -(-E // (NS * ROWLEN))
E_PAD = NS * ROWLEN * KROWS
ACC_ROWS = NSEG + L
ZROWS_TILE = ACC_ROWS // NS
ZCH = ZROWS_TILE // 3
SEG_TILE = NSEG // NS


def _make_sc_agg(with_cnt, interpret=False):
    outs = [jax.ShapeDtypeStruct((NSLICE, NSEG, L), jnp.float32)]
    if with_cnt:
        outs.append(jax.ShapeDtypeStruct((NSEG, L), jnp.float32))
    mesh = plsc.VectorSubcoreMesh(core_axis_name="c", subcore_axis_name="s",
                                  num_cores=NC, num_subcores=NS)

    @functools.partial(
        pl.kernel,
        out_type=tuple(outs) if with_cnt else outs[0],
        mesh=mesh,
        interpret=interpret,
        compiler_params=pltpu.CompilerParams(use_tc_tiling_on_sc=False),
        scratch_types=[
            pltpu.VMEM_SHARED((ACC_ROWS, L), jnp.float32),
            pltpu.VMEM((KROWS, ROWLEN), jnp.int32),
            pltpu.VMEM((KROWS, ROWLEN), jnp.int32),
            pltpu.VMEM((ROWLEN, L), jnp.float32),
            pltpu.VMEM((ROWLEN, L), jnp.float32),
            pltpu.SemaphoreType.DMA,
        ],
    )
    def sc_kernel(xv_hbm, gidx_hbm, seg_hbm, ones_hbm, zeros_hbm, *rest):
        if with_cnt:
            a_out, cnt_out, acc_sh, seg_v, gi_v, row_v, ones_v, gsem = rest
        else:
            a_out, acc_sh, seg_v, gi_v, row_v, ones_v, gsem = rest
        c = lax.axis_index("c")
        s = lax.axis_index("s")

        pltpu.sync_copy(seg_hbm.at[s], seg_v)
        pltpu.sync_copy(ones_hbm, ones_v)

        def zero_my_stripe():
            for kz in range(3):
                pltpu.sync_copy(
                    zeros_hbm, acc_sh.at[pl.ds(s * ZROWS_TILE + kz * ZCH, ZCH)])

        for i in range(NSLICE // NC):
            pglob = c * (NSLICE // NC) + i
            zero_my_stripe()
            plsc.subcore_barrier()
            pltpu.sync_copy(gidx_hbm.at[pglob, s], gi_v)

            def ebody(j, carry):
                pltpu.async_copy(xv_hbm.at[gi_v.at[j]], row_v, gsem).wait()
                pltpu.sync_copy(row_v, acc_sh.at[seg_v.at[j]], add=True)
                return carry

            lax.fori_loop(0, KROWS, ebody, 0)
            plsc.subcore_barrier()
            pltpu.sync_copy(acc_sh.at[pl.ds(s * SEG_TILE, SEG_TILE)],
                            a_out.at[pglob, pl.ds(s * SEG_TILE, SEG_TILE)])
            plsc.subcore_barrier()

        if with_cnt:
            @pl.when(c == 1)
            def _():
                zero_my_stripe()
                plsc.subcore_barrier()

                def cbody(j, carry):
                    pltpu.sync_copy(ones_v, acc_sh.at[seg_v.at[j]], add=True)
                    return carry

                lax.fori_loop(0, KROWS, cbody, 0)
                plsc.subcore_barrier()
                pltpu.sync_copy(acc_sh.at[pl.ds(s * SEG_TILE, SEG_TILE)],
                                cnt_out.at[pl.ds(s * SEG_TILE, SEG_TILE)])

    return sc_kernel


def _tc_layer(a, cnt, xin, basis, comp, root, bias, act, interpret=False):
    out_dim = root.shape[1]
    w = jnp.einsum('rb,bio->rio', comp, basis)
    w2 = w.reshape(R, NSLICE, L, out_dim).transpose(1, 0, 2, 3).reshape(
        NSLICE, F, out_dim)
    a3 = a.reshape(NSLICE, N, F)
    cnt2 = cnt.reshape(N, F)
    bias2 = bias.reshape(1, out_dim)
    nb = 1000
    grid = (N // nb,)

    def body(a_ref, c_ref, x_ref, w2_ref, root_ref, b_ref, o_ref):
        recip = 1.0 / jnp.maximum(c_ref[...], 1.0)
        acc = jnp.dot(x_ref[...], root_ref[...],
                      preferred_element_type=jnp.float32)
        for p in range(NSLICE):
            acc += jnp.dot(a_ref[p] * recip, w2_ref[p],
                           preferred_element_type=jnp.float32)
        acc += b_ref[...]
        if act == 'relu':
            acc = jnp.maximum(acc, 0.0)
        else:
            acc = jnp.tanh(acc)
        o_ref[...] = acc

    return pl.pallas_call(
        body,
        grid=grid,
        in_specs=[
            pl.BlockSpec((NSLICE, nb, F), lambda i: (0, i, 0)),
            pl.BlockSpec((nb, F), lambda i: (i, 0)),
            pl.BlockSpec((nb, F), lambda i: (i, 0)),
            pl.BlockSpec((NSLICE, F, out_dim), lambda i: (0, 0, 0)),
            pl.BlockSpec((F, out_dim), lambda i: (0, 0)),
            pl.BlockSpec((1, out_dim), lambda i: (0, 0)),
        ],
        out_specs=pl.BlockSpec((nb, out_dim), lambda i: (i, 0)),
        out_shape=jax.ShapeDtypeStruct((N, out_dim), jnp.float32),
        interpret=interpret,
    )(a3, cnt2, xin, w2, root, bias2)


def kernel(x, edge_index, edge_type,
           basis0, comp0, root0, bias0,
           basis1, comp1, root1, bias1,
           basis2, comp2, root2, bias2):
    src = edge_index[0].astype(jnp.int32)
    dst = edge_index[1].astype(jnp.int32)
    seg = dst * R + edge_type.astype(jnp.int32)

    pad = E_PAD - E
    seg_p = jnp.concatenate(
        [seg, jnp.full((pad,), NSEG, jnp.int32)]).reshape(NS, KROWS, ROWLEN)
    src_p = jnp.concatenate([src, jnp.zeros((pad,), jnp.int32)])
    src_rs = src_p.reshape(NS, KROWS, ROWLEN)
    gidx = (src_rs[None] * NSLICE
            + jnp.arange(NSLICE, dtype=jnp.int32)[:, None, None, None])
    ones_rows = jnp.ones((ROWLEN, L), jnp.float32)
    zeros_buf = jnp.zeros((ZCH, L), jnp.float32)

    sc_first = _make_sc_agg(True)
    sc_rest = _make_sc_agg(False)

    a0, cnt = sc_first(x.reshape(N * NSLICE, L), gidx, seg_p,
                       ones_rows, zeros_buf)
    h0 = _tc_layer(a0, cnt, x, basis0, comp0, root0, bias0, 'relu')
    a1 = sc_rest(h0.reshape(N * NSLICE, L), gidx, seg_p, ones_rows, zeros_buf)
    h1 = _tc_layer(a1, cnt, h0, basis1, comp1, root1, bias1, 'relu')
    a2 = sc_rest(h1.reshape(N * NSLICE, L), gidx, seg_p, ones_rows, zeros_buf)
    return _tc_layer(a2, cnt, h1, basis2, comp2, root2, bias2, 'tanh')


# speedup vs baseline: 3.5767x; 1.2462x over previous
import functools

import jax
import jax.numpy as jnp
from jax import lax
from jax.experimental import pallas as pl
from jax.experimental.pallas import tpu as pltpu
from jax.experimental.pallas import tpu_sc as plsc

N = 10000
E = 320000
R = 8
F = 128
L = 16
NSLICE = F // L
NSEG = N * R
NC = 2
NS = 16
ROWLEN = 128
NBUF = 8
KROWS = 160
NBLK = KROWS // NBUF
E_PAD = NS * ROWLEN * KROWS
ACC_ROWS = NSEG + L
ZROWS_TILE = ACC_ROWS // NS
ZCH = ZROWS_TILE // 3
SEG_TILE = NSEG // NS


def _make_sc_agg(with_cnt, interpret=False):
    outs = [jax.ShapeDtypeStruct((NSLICE, NSEG, L), jnp.float32)]
    if with_cnt:
        outs.append(jax.ShapeDtypeStruct((NSEG, L), jnp.float32))
    mesh = plsc.VectorSubcoreMesh(core_axis_name="c", subcore_axis_name="s",
                                  num_cores=NC, num_subcores=NS)

    @functools.partial(
        pl.kernel,
        out_type=tuple(outs) if with_cnt else outs[0],
        mesh=mesh,
        interpret=interpret,
        compiler_params=pltpu.CompilerParams(use_tc_tiling_on_sc=False),
        scratch_types=[
            pltpu.VMEM_SHARED((ACC_ROWS, L), jnp.float32),
            pltpu.VMEM((2, NBUF, 2, ROWLEN), jnp.int32),
            pltpu.VMEM((2, NBUF, ROWLEN, L), jnp.float32),
            pltpu.VMEM((ROWLEN, L), jnp.float32),
            pltpu.SemaphoreType.DMA,
            pltpu.SemaphoreType.DMA,
            pltpu.SemaphoreType.DMA,
        ],
    )
    def sc_kernel(xv_hbm, idx2_hbm, ones_hbm, zeros_hbm, *rest):
        if with_cnt:
            (a_out, cnt_out, acc_sh, idx_v, row_v, ones_v,
             gsem, ssem, isem) = rest
        else:
            a_out, acc_sh, idx_v, row_v, ones_v, gsem, ssem, isem = rest
        c = lax.axis_index("c")
        s = lax.axis_index("s")

        pltpu.sync_copy(ones_hbm, ones_v)

        def zero_my_stripe():
            for kz in range(3):
                pltpu.sync_copy(
                    zeros_hbm, acc_sh.at[pl.ds(s * ZROWS_TILE + kz * ZCH, ZCH)])

        def iwait():
            pltpu.make_async_copy(idx2_hbm.at[0, 0, 0], idx_v.at[0],
                                  isem).wait()

        def gissue(h):
            for b in range(NBUF):
                pltpu.async_copy(xv_hbm.at[idx_v.at[h, b, 0]],
                                 row_v.at[h, b], gsem)

        def gdrain():
            for _ in range(NBUF):
                pltpu.make_async_copy(xv_hbm.at[pl.ds(0, ROWLEN)],
                                      row_v.at[0, 0], gsem).wait()

        def sissue(h):
            for b in range(NBUF):
                pltpu.async_copy(row_v.at[h, b],
                                 acc_sh.at[idx_v.at[h, b, 1]], ssem, add=True)

        def sdrain():
            for _ in range(NBUF):
                pltpu.make_async_copy(row_v.at[0, 0],
                                      acc_sh.at[pl.ds(0, ROWLEN)], ssem).wait()

        for i in range(NSLICE // NC):
            pglob = c * (NSLICE // NC) + i

            def istart(blk, h):
                pltpu.async_copy(idx2_hbm.at[pglob, s, blk], idx_v.at[h], isem)

            zero_my_stripe()
            plsc.subcore_barrier()

            istart(0, 0)
            iwait()
            gissue(0)
            istart(1, 1)

            def pairbody(ip, carry):
                for h in range(2):
                    jg = ip * 2 + h
                    gdrain()

                    @pl.when(jg + 1 < NBLK)
                    def _():
                        iwait()

                    sissue(h)

                    @pl.when(jg + 1 < NBLK)
                    def _():
                        gissue(1 - h)

                    sdrain()

                    @pl.when(jg + 2 < NBLK)
                    def _():
                        istart(jg + 2, h)
                return carry

            lax.fori_loop(0, NBLK // 2, pairbody, 0)
            plsc.subcore_barrier()
            pltpu.sync_copy(acc_sh.at[pl.ds(s * SEG_TILE, SEG_TILE)],
                            a_out.at[pglob, pl.ds(s * SEG_TILE, SEG_TILE)])
            plsc.subcore_barrier()

        if with_cnt:
            @pl.when(c == 1)
            def _():
                zero_my_stripe()
                plsc.subcore_barrier()
                pltpu.async_copy(idx2_hbm.at[0, s, 0], idx_v.at[0], isem)

                def cpair(ip, carry):
                    for h in range(2):
                        jg = ip * 2 + h
                        iwait()

                        @pl.when(jg + 1 < NBLK)
                        def _():
                            pltpu.async_copy(idx2_hbm.at[0, s, jg + 1],
                                             idx_v.at[1 - h], isem)

                        for b in range(NBUF):
                            pltpu.async_copy(
                                ones_v, acc_sh.at[idx_v.at[h, b, 1]],
                                ssem, add=True)
                        sdrain()
                    return carry

                lax.fori_loop(0, NBLK // 2, cpair, 0)
                plsc.subcore_barrier()
                pltpu.sync_copy(acc_sh.at[pl.ds(s * SEG_TILE, SEG_TILE)],
                                cnt_out.at[pl.ds(s * SEG_TILE, SEG_TILE)])

    return sc_kernel


def _tc_layer(a, cnt, xin, basis, comp, root, bias, act, interpret=False):
    out_dim = root.shape[1]
    w = jnp.einsum('rb,bio->rio', comp, basis)
    w2 = w.reshape(R, NSLICE, L, out_dim).transpose(1, 0, 2, 3).reshape(
        NSLICE, F, out_dim)
    a3 = a.reshape(NSLICE, N, F)
    cnt2 = cnt.reshape(N, F)
    bias2 = bias.reshape(1, out_dim)
    nb = 1000
    grid = (N // nb,)

    def body(a_ref, c_ref, x_ref, w2_ref, root_ref, b_ref, o_ref):
        recip = 1.0 / jnp.maximum(c_ref[...], 1.0)
        acc = jnp.dot(x_ref[...], root_ref[...],
                      preferred_element_type=jnp.float32)
        for p in range(NSLICE):
            acc += jnp.dot(a_ref[p] * recip, w2_ref[p],
                           preferred_element_type=jnp.float32)
        acc += b_ref[...]
        if act == 'relu':
            acc = jnp.maximum(acc, 0.0)
        else:
            acc = jnp.tanh(acc)
        o_ref[...] = acc

    return pl.pallas_call(
        body,
        grid=grid,
        in_specs=[
            pl.BlockSpec((NSLICE, nb, F), lambda i: (0, i, 0)),
            pl.BlockSpec((nb, F), lambda i: (i, 0)),
            pl.BlockSpec((nb, F), lambda i: (i, 0)),
            pl.BlockSpec((NSLICE, F, out_dim), lambda i: (0, 0, 0)),
            pl.BlockSpec((F, out_dim), lambda i: (0, 0)),
            pl.BlockSpec((1, out_dim), lambda i: (0, 0)),
        ],
        out_specs=pl.BlockSpec((nb, out_dim), lambda i: (i, 0)),
        out_shape=jax.ShapeDtypeStruct((N, out_dim), jnp.float32),
        interpret=interpret,
    )(a3, cnt2, xin, w2, root, bias2)


def kernel(x, edge_index, edge_type,
           basis0, comp0, root0, bias0,
           basis1, comp1, root1, bias1,
           basis2, comp2, root2, bias2):
    src = edge_index[0].astype(jnp.int32)
    dst = edge_index[1].astype(jnp.int32)
    seg = dst * R + edge_type.astype(jnp.int32)

    pad = E_PAD - E
    seg_p = jnp.concatenate(
        [seg, jnp.full((pad,), NSEG, jnp.int32)]).reshape(
            NS, NBLK, NBUF, ROWLEN)
    src_p = jnp.concatenate([src, jnp.zeros((pad,), jnp.int32)])
    src_rs = src_p.reshape(NS, NBLK, NBUF, ROWLEN)
    gidx = (src_rs[None] * NSLICE
            + jnp.arange(NSLICE, dtype=jnp.int32)[:, None, None, None, None])
    idx2 = jnp.stack(
        [gidx, jnp.broadcast_to(seg_p[None], gidx.shape)], axis=4)
    ones_rows = jnp.ones((ROWLEN, L), jnp.float32)
    zeros_buf = jnp.zeros((ZCH, L), jnp.float32)

    sc_first = _make_sc_agg(True)
    sc_rest = _make_sc_agg(False)

    a0, cnt = sc_first(x.reshape(N * NSLICE, L), idx2, ones_rows, zeros_buf)
    h0 = _tc_layer(a0, cnt, x, basis0, comp0, root0, bias0, 'relu')
    a1 = sc_rest(h0.reshape(N * NSLICE, L), idx2, ones_rows, zeros_buf)
    h1 = _tc_layer(a1, cnt, h0, basis1, comp1, root1, bias1, 'relu')
    a2 = sc_rest(h1.reshape(N * NSLICE, L), idx2, ones_rows, zeros_buf)
    return _tc_layer(a2, cnt, h1, basis2, comp2, root2, bias2, 'tanh')


# speedup vs baseline: 5.1550x; 1.4413x over previous
import functools

import jax
import jax.numpy as jnp
from jax import lax
from jax.experimental import pallas as pl
from jax.experimental.pallas import tpu as pltpu
from jax.experimental.pallas import tpu_sc as plsc

N = 10000
E = 320000
R = 8
F = 128
L = 16
LB = 32
NSLICE = F // LB
NSEG = N * R
NC = 2
NS = 16
ROWLEN = 128
NBUF = 8
KROWS = 160
NBLK = KROWS // NBUF
E_PAD = NS * ROWLEN * KROWS
ACC_ROWS = NSEG + L
ZROWS_TILE = ACC_ROWS // NS
ZCH = ZROWS_TILE // 3
SEG_TILE = NSEG // NS
CBLK = NBLK // NC


def _make_sc_agg(interpret=False):
    mesh = plsc.VectorSubcoreMesh(core_axis_name="c", subcore_axis_name="s",
                                  num_cores=NC, num_subcores=NS)

    @functools.partial(
        pl.kernel,
        out_type=jax.ShapeDtypeStruct((NSLICE, NSEG, LB), jnp.bfloat16),
        mesh=mesh,
        interpret=interpret,
        compiler_params=pltpu.CompilerParams(use_tc_tiling_on_sc=False),
        scratch_types=[
            pltpu.VMEM_SHARED((ACC_ROWS, LB), jnp.bfloat16),
            pltpu.VMEM((2, NBUF, 2, ROWLEN), jnp.int32),
            pltpu.VMEM((2, NBUF, ROWLEN, LB), jnp.bfloat16),
            pltpu.SemaphoreType.DMA,
            pltpu.SemaphoreType.DMA,
            pltpu.SemaphoreType.DMA,
        ],
    )
    def sc_kernel(xv_hbm, idx2_hbm, zeros_hbm, a_out,
                  acc_sh, idx_v, row_v, gsem, ssem, isem):
        c = lax.axis_index("c")
        s = lax.axis_index("s")

        def zero_my_stripe():
            for kz in range(3):
                pltpu.sync_copy(
                    zeros_hbm, acc_sh.at[pl.ds(s * ZROWS_TILE + kz * ZCH, ZCH)])

        def iwait():
            pltpu.make_async_copy(idx2_hbm.at[0, 0, 0], idx_v.at[0],
                                  isem).wait()

        def gissue(h):
            for b in range(NBUF):
                pltpu.async_copy(xv_hbm.at[idx_v.at[h, b, 0]],
                                 row_v.at[h, b], gsem)

        def gdrain():
            for _ in range(NBUF):
                pltpu.make_async_copy(xv_hbm.at[pl.ds(0, ROWLEN)],
                                      row_v.at[0, 0], gsem).wait()

        def sissue(h):
            for b in range(NBUF):
                pltpu.async_copy(row_v.at[h, b],
                                 acc_sh.at[idx_v.at[h, b, 1]], ssem, add=True)

        def sdrain():
            for _ in range(NBUF):
                pltpu.make_async_copy(row_v.at[0, 0],
                                      acc_sh.at[pl.ds(0, ROWLEN)], ssem).wait()

        for i in range(NSLICE // NC):
            pglob = c * (NSLICE // NC) + i

            def istart(blk, h):
                pltpu.async_copy(idx2_hbm.at[pglob, s, blk], idx_v.at[h], isem)

            zero_my_stripe()
            plsc.subcore_barrier()

            istart(0, 0)
            iwait()
            gissue(0)
            istart(1, 1)

            def pairbody(ip, carry):
                for h in range(2):
                    jg = ip * 2 + h
                    gdrain()

                    @pl.when(jg + 1 < NBLK)
                    def _():
                        iwait()

                    sissue(h)

                    @pl.when(jg + 1 < NBLK)
                    def _():
                        gissue(1 - h)

                    sdrain()

                    @pl.when(jg + 2 < NBLK)
                    def _():
                        istart(jg + 2, h)
                return carry

            lax.fori_loop(0, NBLK // 2, pairbody, 0)
            plsc.subcore_barrier()
            pltpu.sync_copy(acc_sh.at[pl.ds(s * SEG_TILE, SEG_TILE)],
                            a_out.at[pglob, pl.ds(s * SEG_TILE, SEG_TILE)])
            plsc.subcore_barrier()

    return sc_kernel


def _make_sc_cnt(interpret=False):
    mesh = plsc.VectorSubcoreMesh(core_axis_name="c", subcore_axis_name="s",
                                  num_cores=NC, num_subcores=NS)

    @functools.partial(
        pl.kernel,
        out_type=jax.ShapeDtypeStruct((NC, NSEG, L), jnp.float32),
        mesh=mesh,
        interpret=interpret,
        compiler_params=pltpu.CompilerParams(use_tc_tiling_on_sc=False),
        scratch_types=[
            pltpu.VMEM_SHARED((ACC_ROWS, L), jnp.float32),
            pltpu.VMEM((2, NBUF, 2, ROWLEN), jnp.int32),
            pltpu.VMEM((ROWLEN, L), jnp.float32),
            pltpu.SemaphoreType.DMA,
            pltpu.SemaphoreType.DMA,
        ],
    )
    def cnt_kernel(idx2_hbm, ones_hbm, zeros_hbm, cnt_out,
                   acc_sh, idx_v, ones_v, ssem, isem):
        c = lax.axis_index("c")
        s = lax.axis_index("s")
        blk0 = c * CBLK

        pltpu.sync_copy(ones_hbm, ones_v)
        for kz in range(3):
            pltpu.sync_copy(
                zeros_hbm, acc_sh.at[pl.ds(s * ZROWS_TILE + kz * ZCH, ZCH)])
        plsc.subcore_barrier()

        def iwait():
            pltpu.make_async_copy(idx2_hbm.at[0, 0, 0], idx_v.at[0],
                                  isem).wait()

        def sdrain():
            for _ in range(NBUF):
                pltpu.make_async_copy(ones_v, acc_sh.at[pl.ds(0, ROWLEN)],
                                      ssem).wait()

        pltpu.async_copy(idx2_hbm.at[0, s, blk0], idx_v.at[0], isem)

        def cpair(ip, carry):
            for h in range(2):
                jg = ip * 2 + h
                iwait()

                @pl.when(jg + 1 < CBLK)
                def _():
                    pltpu.async_copy(idx2_hbm.at[0, s, blk0 + jg + 1],
                                     idx_v.at[1 - h], isem)

                for b in range(NBUF):
                    pltpu.async_copy(ones_v, acc_sh.at[idx_v.at[h, b, 1]],
                                     ssem, add=True)
                sdrain()
            return carry

        lax.fori_loop(0, CBLK // 2, cpair, 0)
        plsc.subcore_barrier()
        pltpu.sync_copy(acc_sh.at[pl.ds(s * SEG_TILE, SEG_TILE)],
                        cnt_out.at[c, pl.ds(s * SEG_TILE, SEG_TILE)])

    return cnt_kernel


def _tc_layer(a, cnt0, cnt1, xin, basis, comp, root, bias, act,
              want_bf16, interpret=False):
    out_dim = root.shape[1]
    w = jnp.einsum('rb,bio->rio', comp, basis)
    w2 = w.reshape(R, NSLICE, LB, out_dim).transpose(1, 0, 2, 3).reshape(
        NSLICE, R * LB, out_dim)
    a3 = a.reshape(NSLICE, N, R * LB)
    bias2 = bias.reshape(1, out_dim)
    nb = 1000
    grid = (N // nb,)

    def body(a_ref, c0_ref, c1_ref, x_ref, w2_ref, root_ref, b_ref, *o_refs):
        recip = 1.0 / jnp.maximum(c0_ref[...] + c1_ref[...], 1.0)
        acc = jnp.dot(x_ref[...], root_ref[...],
                      preferred_element_type=jnp.float32)
        for p in range(NSLICE):
            ap = a_ref[p].astype(jnp.float32) * recip
            acc += jnp.dot(ap, w2_ref[p], preferred_element_type=jnp.float32)
        acc += b_ref[...]
        if act == 'relu':
            acc = jnp.maximum(acc, 0.0)
        else:
            acc = jnp.tanh(acc)
        o_refs[0][...] = acc
        if want_bf16:
            o_refs[1][...] = acc.astype(jnp.bfloat16)

    out_shapes = [jax.ShapeDtypeStruct((N, out_dim), jnp.float32)]
    out_specs = [pl.BlockSpec((nb, out_dim), lambda i: (i, 0))]
    if want_bf16:
        out_shapes.append(jax.ShapeDtypeStruct((N, out_dim), jnp.bfloat16))
        out_specs.append(pl.BlockSpec((nb, out_dim), lambda i: (i, 0)))

    return pl.pallas_call(
        body,
        grid=grid,
        in_specs=[
            pl.BlockSpec((NSLICE, nb, R * LB), lambda i: (0, i, 0)),
            pl.BlockSpec((nb, R * LB), lambda i: (i, 0)),
            pl.BlockSpec((nb, R * LB), lambda i: (i, 0)),
            pl.BlockSpec((nb, F), lambda i: (i, 0)),
            pl.BlockSpec((NSLICE, R * LB, out_dim), lambda i: (0, 0, 0)),
            pl.BlockSpec((F, out_dim), lambda i: (0, 0)),
            pl.BlockSpec((1, out_dim), lambda i: (0, 0)),
        ],
        out_specs=out_specs,
        out_shape=out_shapes,
        interpret=interpret,
    )(a3, cnt0, cnt1, xin, w2, root, bias2)


def kernel(x, edge_index, edge_type,
           basis0, comp0, root0, bias0,
           basis1, comp1, root1, bias1,
           basis2, comp2, root2, bias2):
    src = edge_index[0].astype(jnp.int32)
    dst = edge_index[1].astype(jnp.int32)
    seg = dst * R + edge_type.astype(jnp.int32)

    pad = E_PAD - E
    seg_p = jnp.concatenate(
        [seg, jnp.full((pad,), NSEG, jnp.int32)]).reshape(
            NS, NBLK, NBUF, ROWLEN)
    src_p = jnp.concatenate([src, jnp.zeros((pad,), jnp.int32)])
    src_rs = src_p.reshape(NS, NBLK, NBUF, ROWLEN)
    gidx = (src_rs[None] * NSLICE
            + jnp.arange(NSLICE, dtype=jnp.int32)[:, None, None, None, None])
    idx2 = jnp.stack(
        [gidx, jnp.broadcast_to(seg_p[None], gidx.shape)], axis=4)
    ones_rows = jnp.ones((ROWLEN, L), jnp.float32)
    zeros_f32 = jnp.zeros((ZCH, L), jnp.float32)
    zeros_bf = jnp.zeros((ZCH, LB), jnp.bfloat16)

    sc_agg = _make_sc_agg()
    cntp = _make_sc_cnt()(idx2, ones_rows, zeros_f32)
    cnt0 = jnp.repeat(cntp[0, :, 0].reshape(N, R), LB, axis=1)
    cnt1 = jnp.repeat(cntp[1, :, 0].reshape(N, R), LB, axis=1)

    xb = x.astype(jnp.bfloat16)
    a0 = sc_agg(xb.reshape(N * NSLICE, LB), idx2, zeros_bf)
    h0, h0b = _tc_layer(a0, cnt0, cnt1, x, basis0, comp0, root0, bias0,
                        'relu', True)
    a1 = sc_agg(h0b.reshape(N * NSLICE, LB), idx2, zeros_bf)
    h1, h1b = _tc_layer(a1, cnt0, cnt1, h0, basis1, comp1, root1, bias1,
                        'relu', True)
    a2 = sc_agg(h1b.reshape(N * NSLICE, LB), idx2, zeros_bf)
    (out,) = _tc_layer(a2, cnt0, cnt1, h1, basis2, comp2, root2, bias2,
                       'tanh', False)
    return out
